# baseline (speedup 1.0000x reference)
import functools

import jax
import jax.numpy as jnp
from jax import lax
from jax.experimental import pallas as pl
from jax.experimental.pallas import tpu as pltpu
from jax.experimental.pallas import tpu_sc as plsc

FEATURE_DIM = 128
DATA_SIZE = 16384
BATCH = 32
T = 0.07
MOMENTUM = 0.5

ROW_BLK = 4096
NUM_BLKS = DATA_SIZE // ROW_BLK
COPY_BLKS = 2
LANES = 16
UNROLL = 4


def _scores_body(fringe_ref, phase_ref, fea_f_ref, fea_p_ref, y_ref,
                 sf_ref, sp_ref, uf_ref, up_ref, nf_ref, np_ref):
    i = pl.program_id(0)
    fringe = fringe_ref[...]
    phase = phase_ref[...]
    ff = fea_f_ref[...]
    fp = fea_p_ref[...]
    inv_t = jnp.float32(1.0 / T)

    dn_t = (((1,), (1,)), ((), ()))
    sf_ref[...] = lax.dot_general(fp, fringe, dn_t,
                                  preferred_element_type=jnp.float32) * inv_t
    sp_ref[...] = lax.dot_general(ff, phase, dn_t,
                                  preferred_element_type=jnp.float32) * inv_t

    @pl.when(i < COPY_BLKS)
    def _():
        nf_ref[...] = fringe
        np_ref[...] = phase

    y = y_ref[...]
    r = y - i * ROW_BLK
    col = lax.broadcasted_iota(jnp.int32, (BATCH, ROW_BLK), 1)
    onehot = (col == r).astype(jnp.float32)

    @pl.when(i == 0)
    def _():
        uf_ref[...] = (1.0 - MOMENTUM) * ff
        up_ref[...] = (1.0 - MOMENTUM) * fp

    dn_g = (((1,), (0,)), ((), ()))
    uf_ref[...] += MOMENTUM * lax.dot_general(
        onehot, fringe, dn_g, preferred_element_type=jnp.float32)
    up_ref[...] += MOMENTUM * lax.dot_general(
        onehot, phase, dn_g, preferred_element_type=jnp.float32)

    @pl.when(i == NUM_BLKS - 1)
    def _():
        uf = uf_ref[...]
        up = up_ref[...]
        uf_ref[...] = uf / jnp.maximum(
            jnp.sqrt(jnp.sum(uf * uf, axis=1, keepdims=True)), 1e-30)
        up_ref[...] = up / jnp.maximum(
            jnp.sqrt(jnp.sum(up * up, axis=1, keepdims=True)), 1e-30)


def _scores_call(fringe, phase, ff, fp, yv):
    blk = (ROW_BLK, FEATURE_DIM)
    return pl.pallas_call(
        _scores_body,
        grid=(NUM_BLKS,),
        in_specs=[
            pl.BlockSpec(blk, lambda i: (i, 0)),
            pl.BlockSpec(blk, lambda i: (i, 0)),
            pl.BlockSpec((BATCH, FEATURE_DIM), lambda i: (0, 0)),
            pl.BlockSpec((BATCH, FEATURE_DIM), lambda i: (0, 0)),
            pl.BlockSpec((BATCH, 1), lambda i: (0, 0)),
        ],
        out_specs=[
            pl.BlockSpec((BATCH, ROW_BLK), lambda i: (0, i)),
            pl.BlockSpec((BATCH, ROW_BLK), lambda i: (0, i)),
            pl.BlockSpec((BATCH, FEATURE_DIM), lambda i: (0, 0)),
            pl.BlockSpec((BATCH, FEATURE_DIM), lambda i: (0, 0)),
            pl.BlockSpec(blk, lambda i: (jnp.minimum(i, COPY_BLKS - 1), 0)),
            pl.BlockSpec(blk, lambda i: (jnp.minimum(i, COPY_BLKS - 1), 0)),
        ],
        out_shape=[
            jax.ShapeDtypeStruct((BATCH, DATA_SIZE), jnp.float32),
            jax.ShapeDtypeStruct((BATCH, DATA_SIZE), jnp.float32),
            jax.ShapeDtypeStruct((BATCH, FEATURE_DIM), jnp.float32),
            jax.ShapeDtypeStruct((BATCH, FEATURE_DIM), jnp.float32),
            jax.ShapeDtypeStruct((DATA_SIZE, FEATURE_DIM), jnp.float32),
            jax.ShapeDtypeStruct((DATA_SIZE, FEATURE_DIM), jnp.float32),
        ],
        compiler_params=pltpu.CompilerParams(
            dimension_semantics=("arbitrary",),
        ),
    )(fringe, phase, ff, fp, yv)


def _copy_body(fringe_ref, phase_ref, fbuf_ref, pbuf_ref, nf_ref, np_ref):
    del fbuf_ref, pbuf_ref
    nf_ref[...] = fringe_ref[...]
    np_ref[...] = phase_ref[...]


def _copy_call(fringe, phase, fbuf, pbuf):
    blk = (ROW_BLK, FEATURE_DIM)
    rest = NUM_BLKS - COPY_BLKS
    return pl.pallas_call(
        _copy_body,
        grid=(rest,),
        in_specs=[
            pl.BlockSpec(blk, lambda i: (i + COPY_BLKS, 0)),
            pl.BlockSpec(blk, lambda i: (i + COPY_BLKS, 0)),
            pl.BlockSpec(blk, lambda i: (i + COPY_BLKS, 0)),
            pl.BlockSpec(blk, lambda i: (i + COPY_BLKS, 0)),
        ],
        out_specs=[
            pl.BlockSpec(blk, lambda i: (i + COPY_BLKS, 0)),
            pl.BlockSpec(blk, lambda i: (i + COPY_BLKS, 0)),
        ],
        out_shape=[
            jax.ShapeDtypeStruct((DATA_SIZE, FEATURE_DIM), jnp.float32),
            jax.ShapeDtypeStruct((DATA_SIZE, FEATURE_DIM), jnp.float32),
        ],
        input_output_aliases={2: 0, 3: 1},
        compiler_params=pltpu.CompilerParams(
            dimension_semantics=("arbitrary",),
        ),
    )(fringe, phase, fbuf, pbuf)


def _rowscatter_body(y_ref, uf_ref, up_ref, fin_ref, pin_ref,
                     fout_ref, pout_ref):
    del y_ref, fin_ref, pin_ref
    fout_ref[...] = uf_ref[...]
    pout_ref[...] = up_ref[...]


def _rowscatter_call(y, unf, unp, fbuf, pbuf):
    unf3 = unf.reshape(BATCH, 1, FEATURE_DIM)
    unp3 = unp.reshape(BATCH, 1, FEATURE_DIM)
    fbuf3 = fbuf.reshape(DATA_SIZE, 1, FEATURE_DIM)
    pbuf3 = pbuf.reshape(DATA_SIZE, 1, FEATURE_DIM)
    blk = (1, 1, FEATURE_DIM)
    grid_spec = pltpu.PrefetchScalarGridSpec(
        num_scalar_prefetch=1,
        grid=(BATCH,),
        in_specs=[
            pl.BlockSpec(blk, lambda i, y_ref: (i, 0, 0)),
            pl.BlockSpec(blk, lambda i, y_ref: (i, 0, 0)),
            pl.BlockSpec(blk, lambda i, y_ref: (y_ref[i], 0, 0)),
            pl.BlockSpec(blk, lambda i, y_ref: (y_ref[i], 0, 0)),
        ],
        out_specs=[
            pl.BlockSpec(blk, lambda i, y_ref: (y_ref[i], 0, 0)),
            pl.BlockSpec(blk, lambda i, y_ref: (y_ref[i], 0, 0)),
        ],
    )
    nf3, np3 = pl.pallas_call(
        _rowscatter_body,
        grid_spec=grid_spec,
        out_shape=[
            jax.ShapeDtypeStruct((DATA_SIZE, 1, FEATURE_DIM), jnp.float32),
            jax.ShapeDtypeStruct((DATA_SIZE, 1, FEATURE_DIM), jnp.float32),
        ],
        input_output_aliases={3: 0, 4: 1},
        compiler_params=pltpu.CompilerParams(
            dimension_semantics=("arbitrary",),
        ),
    )(y, unf3, unp3, fbuf3, pbuf3)
    return (nf3.reshape(DATA_SIZE, FEATURE_DIM),
            np3.reshape(DATA_SIZE, FEATURE_DIM))


def _sc_gather(idx, sf, sp):
    mesh = plsc.VectorSubcoreMesh(core_axis_name="c", subcore_axis_name="s")
    info = plsc.get_sparse_core_info()
    n_cores = info.num_cores

    @functools.partial(
        pl.kernel,
        mesh=mesh,
        out_type=[
            jax.ShapeDtypeStruct((BATCH, DATA_SIZE), jnp.float32),
            jax.ShapeDtypeStruct((BATCH, DATA_SIZE), jnp.float32),
        ],
        scratch_types=[
            pltpu.VMEM((DATA_SIZE,), jnp.int32),
            pltpu.VMEM((DATA_SIZE,), jnp.float32),
            pltpu.VMEM((DATA_SIZE,), jnp.float32),
            pltpu.VMEM((DATA_SIZE,), jnp.float32),
            pltpu.VMEM((DATA_SIZE,), jnp.float32),
            pltpu.SemaphoreType.DMA,
        ],
        compiler_params=pltpu.CompilerParams(needs_layout_passes=False),
    )
    def k(idx_hbm, sf_hbm, sp_hbm, outp_hbm, outf_hbm,
          idx_v, sf_v, sp_v, outp_v, outf_v, sem):
        wid = lax.axis_index("s") * n_cores + lax.axis_index("c")
        c1 = pltpu.async_copy(idx_hbm.at[wid], idx_v, sem)
        c2 = pltpu.async_copy(sf_hbm.at[wid], sf_v, sem)
        c3 = pltpu.async_copy(sp_hbm.at[wid], sp_v, sem)
        c1.wait()
        c2.wait()
        c3.wait()

        def body(j, carry):
            base = j * (LANES * UNROLL)
            for u in range(UNROLL):
                o = base + u * LANES
                v_idx = idx_v[pl.ds(o, LANES)]
                outp_v[pl.ds(o, LANES)] = plsc.load_gather(sf_v, [v_idx])
                outf_v[pl.ds(o, LANES)] = plsc.load_gather(sp_v, [v_idx])
            return carry

        lax.fori_loop(0, DATA_SIZE // (LANES * UNROLL), body, 0)
        c4 = pltpu.async_copy(outp_v, outp_hbm.at[wid], sem)
        c5 = pltpu.async_copy(outf_v, outf_hbm.at[wid], sem)
        c4.wait()
        c5.wait()

    return k(idx, sf, sp)


def kernel(fea_f, fea_p, y, idx, memory_fringe, memory_phase):
    y32 = y.astype(jnp.int32)
    yv = y32.reshape(BATCH, 1)
    idx32 = idx.astype(jnp.int32)
    sf, sp, unf, unp, fbuf, pbuf = _scores_call(
        memory_fringe, memory_phase, fea_f, fea_p, yv)
    out_phase, out_fringe = _sc_gather(idx32, sf, sp)
    fbuf, pbuf = _copy_call(memory_fringe, memory_phase, fbuf, pbuf)
    new_fringe, new_phase = _rowscatter_call(y32, unf, unp, fbuf, pbuf)
    return (out_fringe.reshape(BATCH, DATA_SIZE, 1),
            out_phase.reshape(BATCH, DATA_SIZE, 1),
            new_fringe, new_phase)


# speedup vs baseline: 1.2773x; 1.2773x over previous
import functools

import jax
import jax.numpy as jnp
from jax import lax
from jax.experimental import pallas as pl
from jax.experimental.pallas import tpu as pltpu
from jax.experimental.pallas import tpu_sc as plsc

FEATURE_DIM = 128
DATA_SIZE = 16384
BATCH = 32
T = 0.07
MOMENTUM = 0.5

ROW_BLK = 8192
NUM_BLKS = DATA_SIZE // ROW_BLK
LANES = 16
UNROLL = 4


def _scores_body(fringe_ref, phase_ref, fea_f_ref, fea_p_ref, y_ref,
                 sf_ref, sp_ref, uf_ref, up_ref):
    i = pl.program_id(0)
    fringe = fringe_ref[...]
    phase = phase_ref[...]
    ff = fea_f_ref[...]
    fp = fea_p_ref[...]
    inv_t = jnp.float32(1.0 / T)

    dn_t = (((1,), (1,)), ((), ()))
    sf_ref[...] = lax.dot_general(fp, fringe, dn_t,
                                  preferred_element_type=jnp.float32) * inv_t
    sp_ref[...] = lax.dot_general(ff, phase, dn_t,
                                  preferred_element_type=jnp.float32) * inv_t

    y = y_ref[...]
    r = y - i * ROW_BLK
    col = lax.broadcasted_iota(jnp.int32, (BATCH, ROW_BLK), 1)
    onehot = (col == r).astype(jnp.float32)

    @pl.when(i == 0)
    def _():
        uf_ref[...] = (1.0 - MOMENTUM) * ff
        up_ref[...] = (1.0 - MOMENTUM) * fp

    dn_g = (((1,), (0,)), ((), ()))
    uf_ref[...] += MOMENTUM * lax.dot_general(
        onehot, fringe, dn_g, preferred_element_type=jnp.float32)
    up_ref[...] += MOMENTUM * lax.dot_general(
        onehot, phase, dn_g, preferred_element_type=jnp.float32)

    @pl.when(i == NUM_BLKS - 1)
    def _():
        uf = uf_ref[...]
        up = up_ref[...]
        uf_ref[...] = uf / jnp.maximum(
            jnp.sqrt(jnp.sum(uf * uf, axis=1, keepdims=True)), 1e-30)
        up_ref[...] = up / jnp.maximum(
            jnp.sqrt(jnp.sum(up * up, axis=1, keepdims=True)), 1e-30)


def _scores_call(fringe, phase, ff, fp, yv):
    blk = (ROW_BLK, FEATURE_DIM)
    return pl.pallas_call(
        _scores_body,
        grid=(NUM_BLKS,),
        in_specs=[
            pl.BlockSpec(blk, lambda i: (i, 0)),
            pl.BlockSpec(blk, lambda i: (i, 0)),
            pl.BlockSpec((BATCH, FEATURE_DIM), lambda i: (0, 0)),
            pl.BlockSpec((BATCH, FEATURE_DIM), lambda i: (0, 0)),
            pl.BlockSpec((BATCH, 1), lambda i: (0, 0)),
        ],
        out_specs=[
            pl.BlockSpec((BATCH, ROW_BLK), lambda i: (0, i)),
            pl.BlockSpec((BATCH, ROW_BLK), lambda i: (0, i)),
            pl.BlockSpec((BATCH, FEATURE_DIM), lambda i: (0, 0)),
            pl.BlockSpec((BATCH, FEATURE_DIM), lambda i: (0, 0)),
        ],
        out_shape=[
            jax.ShapeDtypeStruct((BATCH, DATA_SIZE), jnp.float32),
            jax.ShapeDtypeStruct((BATCH, DATA_SIZE), jnp.float32),
            jax.ShapeDtypeStruct((BATCH, FEATURE_DIM), jnp.float32),
            jax.ShapeDtypeStruct((BATCH, FEATURE_DIM), jnp.float32),
        ],
        compiler_params=pltpu.CompilerParams(
            dimension_semantics=("arbitrary",),
        ),
    )(fringe, phase, ff, fp, yv)


def _update_body(fringe_ref, phase_ref, unf_ref, unp_ref, y_ref,
                 nf_ref, np_ref):
    i = pl.program_id(0)
    fringe = fringe_ref[...]
    phase = phase_ref[...]
    unf = unf_ref[...]
    unp = unp_ref[...]
    y = y_ref[...]
    r = y - i * ROW_BLK
    col = lax.broadcasted_iota(jnp.int32, (BATCH, ROW_BLK), 1)
    onehot = (col == r).astype(jnp.float32)

    yrow = jnp.reshape(y, (1, BATCH))
    eq = y == yrow
    later = (lax.broadcasted_iota(jnp.int32, (BATCH, BATCH), 1)
             > lax.broadcasted_iota(jnp.int32, (BATCH, BATCH), 0))
    dup_later = jnp.any(eq & later, axis=1, keepdims=True)
    oh = onehot * jnp.where(dup_later, 0.0, 1.0)

    dn_s = (((0,), (0,)), ((), ()))
    scat_f = lax.dot_general(oh, unf, dn_s, preferred_element_type=jnp.float32)
    scat_p = lax.dot_general(oh, unp, dn_s, preferred_element_type=jnp.float32)
    ones = jnp.ones((BATCH, FEATURE_DIM), jnp.float32)
    rowcnt = lax.dot_general(oh, ones, dn_s,
                             preferred_element_type=jnp.float32)
    nf_ref[...] = fringe * (1.0 - rowcnt) + scat_f
    np_ref[...] = phase * (1.0 - rowcnt) + scat_p


def _update_call(fringe, phase, unf, unp, yv):
    blk = (ROW_BLK, FEATURE_DIM)
    return pl.pallas_call(
        _update_body,
        grid=(NUM_BLKS,),
        in_specs=[
            pl.BlockSpec(blk, lambda i: (i, 0)),
            pl.BlockSpec(blk, lambda i: (i, 0)),
            pl.BlockSpec((BATCH, FEATURE_DIM), lambda i: (0, 0)),
            pl.BlockSpec((BATCH, FEATURE_DIM), lambda i: (0, 0)),
            pl.BlockSpec((BATCH, 1), lambda i: (0, 0)),
        ],
        out_specs=[
            pl.BlockSpec(blk, lambda i: (i, 0)),
            pl.BlockSpec(blk, lambda i: (i, 0)),
        ],
        out_shape=[
            jax.ShapeDtypeStruct((DATA_SIZE, FEATURE_DIM), jnp.float32),
            jax.ShapeDtypeStruct((DATA_SIZE, FEATURE_DIM), jnp.float32),
        ],
        compiler_params=pltpu.CompilerParams(
            dimension_semantics=("arbitrary",),
        ),
    )(fringe, phase, unf, unp, yv)


def _sc_gather(idx, sf, sp):
    mesh = plsc.VectorSubcoreMesh(core_axis_name="c", subcore_axis_name="s")
    info = plsc.get_sparse_core_info()
    n_cores = info.num_cores

    @functools.partial(
        pl.kernel,
        mesh=mesh,
        out_type=[
            jax.ShapeDtypeStruct((BATCH, DATA_SIZE), jnp.float32),
            jax.ShapeDtypeStruct((BATCH, DATA_SIZE), jnp.float32),
        ],
        scratch_types=[
            pltpu.VMEM((DATA_SIZE,), jnp.int32),
            pltpu.VMEM((DATA_SIZE,), jnp.float32),
            pltpu.VMEM((DATA_SIZE,), jnp.float32),
            pltpu.VMEM((DATA_SIZE,), jnp.float32),
            pltpu.VMEM((DATA_SIZE,), jnp.float32),
            pltpu.SemaphoreType.DMA,
        ],
        compiler_params=pltpu.CompilerParams(needs_layout_passes=False),
    )
    def k(idx_hbm, sf_hbm, sp_hbm, outp_hbm, outf_hbm,
          idx_v, sf_v, sp_v, outp_v, outf_v, sem):
        wid = lax.axis_index("s") * n_cores + lax.axis_index("c")
        c1 = pltpu.async_copy(idx_hbm.at[wid], idx_v, sem)
        c2 = pltpu.async_copy(sf_hbm.at[wid], sf_v, sem)
        c3 = pltpu.async_copy(sp_hbm.at[wid], sp_v, sem)
        c1.wait()
        c2.wait()
        c3.wait()

        def body(j, carry):
            base = j * (LANES * UNROLL)
            for u in range(UNROLL):
                o = base + u * LANES
                v_idx = idx_v[pl.ds(o, LANES)]
                outp_v[pl.ds(o, LANES)] = plsc.load_gather(sf_v, [v_idx])
                outf_v[pl.ds(o, LANES)] = plsc.load_gather(sp_v, [v_idx])
            return carry

        lax.fori_loop(0, DATA_SIZE // (LANES * UNROLL), body, 0)
        c4 = pltpu.async_copy(outp_v, outp_hbm.at[wid], sem)
        c5 = pltpu.async_copy(outf_v, outf_hbm.at[wid], sem)
        c4.wait()
        c5.wait()

    return k(idx, sf, sp)


def kernel(fea_f, fea_p, y, idx, memory_fringe, memory_phase):
    yv = y.astype(jnp.int32).reshape(BATCH, 1)
    idx32 = idx.astype(jnp.int32)
    sf, sp, unf, unp = _scores_call(
        memory_fringe, memory_phase, fea_f, fea_p, yv)
    out_phase, out_fringe = _sc_gather(idx32, sf, sp)
    new_fringe, new_phase = _update_call(
        memory_fringe, memory_phase, unf, unp, yv)
    return (out_fringe.reshape(BATCH, DATA_SIZE, 1),
            out_phase.reshape(BATCH, DATA_SIZE, 1),
            new_fringe, new_phase)
